# Initial kernel scaffold; baseline (speedup 1.0000x reference)
import jax
import jax.numpy as jnp
from jax.experimental import pallas as pl


def kernel(x, w, b, gamma, beta):
    raise NotImplementedError("write your pallas kernel here")


# speedup vs baseline: 1.2782x; 1.2782x over previous
import jax
import jax.numpy as jnp
from jax.experimental import pallas as pl
from jax.experimental.pallas import tpu as pltpu


def _parity_weights(w):
    A = jnp.array([[[1.0, 0.0, 0.0], [0.0, 1.0, 1.0]],
                   [[1.0, 1.0, 0.0], [0.0, 0.0, 1.0]]],
                  dtype=jnp.float32)
    wc = jnp.einsum("pik,qjl,klcd->pqijcd", A, A, w.astype(jnp.float32))
    cin, cout = w.shape[2], w.shape[3]
    return wc.reshape(4, 4 * cin, cout)


def _patch(xbf, ph, pw, TH, W, Cin):
    parts = [xbf[ph + dh:ph + dh + TH, pw + dw:pw + dw + W, :]
             for dh in range(2) for dw in range(2)]
    return jnp.concatenate(parts, axis=-1).reshape(TH * W, 4 * Cin)


def _stats_kernel(xb_ref, xt0_ref, xt1_ref, wc_ref, s_ref, ss_ref):
    TH = xb_ref.shape[1]
    W = xb_ref.shape[2] - 2
    Cin = xb_ref.shape[3]
    Cout = wc_ref.shape[2]

    xloc = jnp.concatenate([xb_ref[0], xt0_ref[0], xt1_ref[0]], axis=0)
    xbf = xloc.astype(jnp.bfloat16)
    s_tot = jnp.zeros((1, Cout), jnp.float32)
    ss_tot = jnp.zeros((1, Cout), jnp.float32)
    for p in range(4):
        ph, pw = p // 2, p % 2
        acc = jnp.dot(_patch(xbf, ph, pw, TH, W, Cin), wc_ref[p],
                      preferred_element_type=jnp.float32)
        s_tot = s_tot + jnp.sum(acc, axis=0, keepdims=True)
        ss_tot = ss_tot + jnp.sum(acc * acc, axis=0, keepdims=True)
    s_ref[...] = s_tot[None]
    ss_ref[...] = ss_tot[None]


def _apply_kernel(xb_ref, xt0_ref, xt1_ref, wc_ref, scale_ref, shift_ref,
                  y_ref):
    TH = xb_ref.shape[1]
    W = xb_ref.shape[2] - 2
    Cin = xb_ref.shape[3]
    Cout = wc_ref.shape[2]

    xloc = jnp.concatenate([xb_ref[0], xt0_ref[0], xt1_ref[0]], axis=0)
    xbf = xloc.astype(jnp.bfloat16)
    scale = scale_ref[...]
    shift = shift_ref[...]
    for p in range(4):
        ph, pw = p // 2, p % 2
        acc = jnp.dot(_patch(xbf, ph, pw, TH, W, Cin), wc_ref[p],
                      preferred_element_type=jnp.float32)
        z = jnp.maximum(acc * scale + shift, 0.0)
        y_ref[0, p] = z.reshape(TH, W, Cout)


def _row_specs(TH, W, Cin):
    return [
        pl.BlockSpec((1, TH, W + 2, Cin), lambda n, r: (n, r, 0, 0)),
        pl.BlockSpec((1, 1, W + 2, Cin), lambda n, r: (n, (r + 1) * TH, 0, 0)),
        pl.BlockSpec((1, 1, W + 2, Cin),
                     lambda n, r: (n, (r + 1) * TH + 1, 0, 0)),
    ]


def kernel(x, w, b, gamma, beta, eps=1e-5):
    del b
    N, Cin, H, W = x.shape
    Cout = w.shape[-1]
    TH = 8
    RT = H // TH

    x_nhwc = jnp.transpose(x, (0, 2, 3, 1)).astype(jnp.float32)
    x_p = jnp.pad(x_nhwc, ((0, 0), (1, 1), (1, 1), (0, 0)))
    wc = _parity_weights(w).astype(jnp.bfloat16)

    s, ss = pl.pallas_call(
        _stats_kernel,
        out_shape=(
            jax.ShapeDtypeStruct((N * RT, 1, Cout), jnp.float32),
            jax.ShapeDtypeStruct((N * RT, 1, Cout), jnp.float32),
        ),
        grid=(N, RT),
        in_specs=_row_specs(TH, W, Cin) + [
            pl.BlockSpec((4, 4 * Cin, Cout), lambda n, r: (0, 0, 0)),
        ],
        out_specs=(
            pl.BlockSpec((1, 1, Cout), lambda n, r: (n * RT + r, 0, 0)),
            pl.BlockSpec((1, 1, Cout), lambda n, r: (n * RT + r, 0, 0)),
        ),
        compiler_params=pltpu.CompilerParams(
            dimension_semantics=("parallel", "parallel"),
            vmem_limit_bytes=64 * 1024 * 1024),
    )(x_p, x_p, x_p, wc)

    count = N * 4 * H * W
    total = jnp.sum(s, axis=0)[0]
    total_sq = jnp.sum(ss, axis=0)[0]
    mean = total / count
    var = jnp.maximum(total_sq / count - mean * mean, 0.0)
    scale = gamma / jnp.sqrt(var + eps)
    shift = beta - mean * scale

    y = pl.pallas_call(
        _apply_kernel,
        out_shape=jax.ShapeDtypeStruct((N, 4, H, W, Cout), jnp.float32),
        grid=(N, RT),
        in_specs=_row_specs(TH, W, Cin) + [
            pl.BlockSpec((4, 4 * Cin, Cout), lambda n, r: (0, 0, 0)),
            pl.BlockSpec((1, Cout), lambda n, r: (0, 0)),
            pl.BlockSpec((1, Cout), lambda n, r: (0, 0)),
        ],
        out_specs=pl.BlockSpec((1, 4, TH, W, Cout), lambda n, r: (n, 0, r, 0, 0)),
        compiler_params=pltpu.CompilerParams(
            dimension_semantics=("parallel", "parallel"),
            vmem_limit_bytes=64 * 1024 * 1024),
    )(x_p, x_p, x_p, wc, scale.reshape(1, Cout), shift.reshape(1, Cout))

    y6 = y.reshape(N, 2, 2, H, W, Cout)
    out = jnp.transpose(y6, (0, 5, 3, 1, 4, 2)).reshape(N, Cout, 2 * H, 2 * W)
    return out


# speedup vs baseline: 1.9116x; 1.4956x over previous
import jax
import jax.numpy as jnp
from jax.experimental import pallas as pl
from jax.experimental.pallas import tpu as pltpu


def _parity_weights(w):
    A = jnp.array([[[1.0, 0.0, 0.0], [0.0, 1.0, 1.0]],
                   [[1.0, 1.0, 0.0], [0.0, 0.0, 1.0]]],
                  dtype=jnp.float32)
    wc = jnp.einsum("pik,qjl,klcd->pqijcd", A, A, w.astype(jnp.float32))
    cin, cout = w.shape[2], w.shape[3]
    return wc.reshape(4, 4 * cin, cout)


def _patch(xbf, ph, pw, TH, W, Cin):
    parts = [xbf[ph + dh:ph + dh + TH, pw + dw:pw + dw + W, :]
             for dh in range(2) for dw in range(2)]
    return jnp.concatenate(parts, axis=-1).reshape(TH * W, 4 * Cin)


def _stats_kernel(xb_ref, xt0_ref, xt1_ref, wc_ref, s_ref, ss_ref):
    TH = xb_ref.shape[1]
    W = xb_ref.shape[2] - 2
    Cin = xb_ref.shape[3]
    Cout = wc_ref.shape[2]

    xloc = jnp.concatenate([xb_ref[0], xt0_ref[0], xt1_ref[0]], axis=0)
    xbf = xloc.astype(jnp.bfloat16)
    s_tot = jnp.zeros((1, Cout), jnp.float32)
    ss_tot = jnp.zeros((1, Cout), jnp.float32)
    for p in range(4):
        ph, pw = p // 2, p % 2
        acc = jnp.dot(_patch(xbf, ph, pw, TH, W, Cin), wc_ref[p],
                      preferred_element_type=jnp.float32)
        s_tot = s_tot + jnp.sum(acc, axis=0, keepdims=True)
        ss_tot = ss_tot + jnp.sum(acc * acc, axis=0, keepdims=True)
    s_ref[...] = s_tot[None]
    ss_ref[...] = ss_tot[None]


def _apply_kernel(xb_ref, xt0_ref, xt1_ref, wc_ref, scale_ref, shift_ref,
                  o_ref, scr):
    TH = xb_ref.shape[1]
    W = xb_ref.shape[2] - 2
    Cin = xb_ref.shape[3]
    Cout = wc_ref.shape[2]

    xloc = jnp.concatenate([xb_ref[0], xt0_ref[0], xt1_ref[0]], axis=0)
    xbf = xloc.astype(jnp.bfloat16)
    scale = scale_ref[...]
    shift = shift_ref[...]
    for p in range(4):
        ph, pw = p // 2, p % 2
        acc = jnp.dot(_patch(xbf, ph, pw, TH, W, Cin), wc_ref[p],
                      preferred_element_type=jnp.float32)
        z = jnp.maximum(acc * scale + shift, 0.0)
        scr[pl.ds(ph, TH, 2), pl.ds(pw, W, 2), :] = z.reshape(TH, W, Cout)
    for r in range(2 * TH):
        o_ref[0, :, r, :] = jnp.transpose(scr[r], (1, 0))


def _row_specs(TH, W, Cin):
    return [
        pl.BlockSpec((1, TH, W + 2, Cin), lambda n, r: (n, r, 0, 0)),
        pl.BlockSpec((1, 1, W + 2, Cin), lambda n, r: (n, (r + 1) * TH, 0, 0)),
        pl.BlockSpec((1, 1, W + 2, Cin),
                     lambda n, r: (n, (r + 1) * TH + 1, 0, 0)),
    ]


def kernel(x, w, b, gamma, beta, eps=1e-5):
    del b
    N, Cin, H, W = x.shape
    Cout = w.shape[-1]
    TH = 8
    RT = H // TH

    x_nhwc = jnp.transpose(x, (0, 2, 3, 1)).astype(jnp.float32)
    x_p = jnp.pad(x_nhwc, ((0, 0), (1, 1), (1, 1), (0, 0)))
    wc = _parity_weights(w).astype(jnp.bfloat16)

    s, ss = pl.pallas_call(
        _stats_kernel,
        out_shape=(
            jax.ShapeDtypeStruct((N * RT, 1, Cout), jnp.float32),
            jax.ShapeDtypeStruct((N * RT, 1, Cout), jnp.float32),
        ),
        grid=(N, RT),
        in_specs=_row_specs(TH, W, Cin) + [
            pl.BlockSpec((4, 4 * Cin, Cout), lambda n, r: (0, 0, 0)),
        ],
        out_specs=(
            pl.BlockSpec((1, 1, Cout), lambda n, r: (n * RT + r, 0, 0)),
            pl.BlockSpec((1, 1, Cout), lambda n, r: (n * RT + r, 0, 0)),
        ),
        compiler_params=pltpu.CompilerParams(
            dimension_semantics=("parallel", "parallel"),
            vmem_limit_bytes=64 * 1024 * 1024),
    )(x_p, x_p, x_p, wc)

    count = N * 4 * H * W
    total = jnp.sum(s, axis=0)[0]
    total_sq = jnp.sum(ss, axis=0)[0]
    mean = total / count
    var = jnp.maximum(total_sq / count - mean * mean, 0.0)
    scale = gamma / jnp.sqrt(var + eps)
    shift = beta - mean * scale

    out = pl.pallas_call(
        _apply_kernel,
        out_shape=jax.ShapeDtypeStruct((N, Cout, 2 * H, 2 * W), jnp.float32),
        grid=(N, RT),
        in_specs=_row_specs(TH, W, Cin) + [
            pl.BlockSpec((4, 4 * Cin, Cout), lambda n, r: (0, 0, 0)),
            pl.BlockSpec((1, Cout), lambda n, r: (0, 0)),
            pl.BlockSpec((1, Cout), lambda n, r: (0, 0)),
        ],
        out_specs=pl.BlockSpec((1, Cout, 2 * TH, 2 * W),
                               lambda n, r: (n, 0, r, 0)),
        scratch_shapes=[pltpu.VMEM((2 * TH, 2 * W, Cout), jnp.float32)],
        compiler_params=pltpu.CompilerParams(
            dimension_semantics=("parallel", "parallel"),
            vmem_limit_bytes=64 * 1024 * 1024),
    )(x_p, x_p, x_p, wc, scale.reshape(1, Cout), shift.reshape(1, Cout))
    return out


# speedup vs baseline: 2.3837x; 1.2469x over previous
import jax
import jax.numpy as jnp
from jax.experimental import pallas as pl
from jax.experimental.pallas import tpu as pltpu


def _parity_weights(w):
    A = jnp.array([[[1.0, 0.0, 0.0], [0.0, 1.0, 1.0]],
                   [[1.0, 1.0, 0.0], [0.0, 0.0, 1.0]]],
                  dtype=jnp.float32)
    wc = jnp.einsum("pik,qjl,klcd->pqijcd", A, A, w.astype(jnp.float32))
    cin, cout = w.shape[2], w.shape[3]
    return wc.reshape(4, 4 * cin, cout)


def _patch(xbf, ph, pw, TH, W, Cin):
    parts = [xbf[ph + dh:ph + dh + TH, pw + dw:pw + dw + W, :]
             for dh in range(2) for dw in range(2)]
    return jnp.concatenate(parts, axis=-1).reshape(TH * W, 4 * Cin)


def _stats_kernel(x_ref, wc_ref, s_ref, ss_ref, *, TH):
    Hp = x_ref.shape[1]
    W = x_ref.shape[2] - 2
    Cin = x_ref.shape[3]
    Cout = wc_ref.shape[2]
    RT = (Hp - 2) // TH

    s_tot = jnp.zeros((1, Cout), jnp.float32)
    ss_tot = jnp.zeros((1, Cout), jnp.float32)
    for r in range(RT):
        xbf = x_ref[0, r * TH:r * TH + TH + 2].astype(jnp.bfloat16)
        for p in range(4):
            ph, pw = p // 2, p % 2
            acc = jnp.dot(_patch(xbf, ph, pw, TH, W, Cin), wc_ref[p],
                          preferred_element_type=jnp.float32)
            s_tot = s_tot + jnp.sum(acc, axis=0, keepdims=True)
            ss_tot = ss_tot + jnp.sum(acc * acc, axis=0, keepdims=True)
    s_ref[...] = s_tot[None]
    ss_ref[...] = ss_tot[None]


def _apply_kernel(x_ref, wc_ref, scale_ref, shift_ref, o_ref, scr, *, TH):
    W = x_ref.shape[2] - 2
    Cin = x_ref.shape[3]
    Cout = wc_ref.shape[2]
    r = pl.program_id(1)

    xbf = x_ref[0, pl.ds(r * TH, TH + 2)].astype(jnp.bfloat16)
    scale = scale_ref[...]
    shift = shift_ref[...]
    for p in range(4):
        ph, pw = p // 2, p % 2
        acc = jnp.dot(_patch(xbf, ph, pw, TH, W, Cin), wc_ref[p],
                      preferred_element_type=jnp.float32)
        z = jnp.maximum(acc * scale + shift, 0.0)
        scr[pl.ds(ph, TH, 2), pl.ds(pw, W, 2), :] = z.reshape(TH, W, Cout)
    for rr in range(2 * TH):
        o_ref[0, :, rr, :] = jnp.transpose(scr[rr], (1, 0))


def kernel(x, w, b, gamma, beta, eps=1e-5):
    del b
    N, Cin, H, W = x.shape
    Cout = w.shape[-1]
    TH = 16
    RT = H // TH

    x_nhwc = jnp.transpose(x, (0, 2, 3, 1)).astype(jnp.float32)
    x_p = jnp.pad(x_nhwc, ((0, 0), (1, 1), (1, 1), (0, 0)))
    wc = _parity_weights(w).astype(jnp.bfloat16)

    import functools
    s, ss = pl.pallas_call(
        functools.partial(_stats_kernel, TH=TH),
        out_shape=(
            jax.ShapeDtypeStruct((N, 1, Cout), jnp.float32),
            jax.ShapeDtypeStruct((N, 1, Cout), jnp.float32),
        ),
        grid=(N,),
        in_specs=[
            pl.BlockSpec((1, H + 2, W + 2, Cin), lambda n: (n, 0, 0, 0)),
            pl.BlockSpec((4, 4 * Cin, Cout), lambda n: (0, 0, 0)),
        ],
        out_specs=(
            pl.BlockSpec((1, 1, Cout), lambda n: (n, 0, 0)),
            pl.BlockSpec((1, 1, Cout), lambda n: (n, 0, 0)),
        ),
        compiler_params=pltpu.CompilerParams(
            dimension_semantics=("parallel",),
            vmem_limit_bytes=64 * 1024 * 1024),
    )(x_p, wc)

    count = N * 4 * H * W
    total = jnp.sum(s, axis=0)[0]
    total_sq = jnp.sum(ss, axis=0)[0]
    mean = total / count
    var = jnp.maximum(total_sq / count - mean * mean, 0.0)
    scale = gamma / jnp.sqrt(var + eps)
    shift = beta - mean * scale

    out = pl.pallas_call(
        functools.partial(_apply_kernel, TH=TH),
        out_shape=jax.ShapeDtypeStruct((N, Cout, 2 * H, 2 * W), jnp.float32),
        grid=(N, RT),
        in_specs=[
            pl.BlockSpec((1, H + 2, W + 2, Cin), lambda n, r: (n, 0, 0, 0)),
            pl.BlockSpec((4, 4 * Cin, Cout), lambda n, r: (0, 0, 0)),
            pl.BlockSpec((1, Cout), lambda n, r: (0, 0)),
            pl.BlockSpec((1, Cout), lambda n, r: (0, 0)),
        ],
        out_specs=pl.BlockSpec((1, Cout, 2 * TH, 2 * W),
                               lambda n, r: (n, 0, r, 0)),
        scratch_shapes=[pltpu.VMEM((2 * TH, 2 * W, Cout), jnp.float32)],
        compiler_params=pltpu.CompilerParams(
            dimension_semantics=("parallel", "arbitrary"),
            vmem_limit_bytes=64 * 1024 * 1024),
    )(x_p, wc, scale.reshape(1, Cout), shift.reshape(1, Cout))
    return out


# speedup vs baseline: 2.3891x; 1.0023x over previous
import functools

import jax
import jax.numpy as jnp
from jax.experimental import pallas as pl
from jax.experimental.pallas import tpu as pltpu


def _parity_weights(w):
    A = jnp.array([[[1.0, 0.0, 0.0], [0.0, 1.0, 1.0]],
                   [[1.0, 1.0, 0.0], [0.0, 0.0, 1.0]]],
                  dtype=jnp.float32)
    wc = jnp.einsum("pik,qjl,klcd->pqijcd", A, A, w.astype(jnp.float32))
    cin, cout = w.shape[2], w.shape[3]
    return wc.reshape(4, 4 * cin, cout)


def _patch(xbf, ph, pw, TH, W, Cin):
    parts = [xbf[ph + dh:ph + dh + TH, pw + dw:pw + dw + W, :]
             for dh in range(2) for dw in range(2)]
    return jnp.concatenate(parts, axis=-1).reshape(TH * W, 4 * Cin)


def _pack_rows(z0, z1):
    lo = jax.lax.bitcast_convert_type(z0, jnp.uint16).astype(jnp.uint32)
    hi = jax.lax.bitcast_convert_type(z1, jnp.uint16).astype(jnp.uint32)
    packed = (lo | (hi << 16)).astype(jnp.int32)
    return pltpu.bitcast(packed, jnp.bfloat16)


def _stats_kernel(x_ref, wc_ref, s_ref, ss_ref, *, TH):
    NB = x_ref.shape[0]
    Hp = x_ref.shape[1]
    W = x_ref.shape[2] - 2
    Cin = x_ref.shape[3]
    Cout = wc_ref.shape[2]
    RT = (Hp - 2) // TH

    for b in range(NB):
        s_tot = jnp.zeros((1, Cout), jnp.float32)
        ss_tot = jnp.zeros((1, Cout), jnp.float32)
        for r in range(RT):
            xbf = x_ref[b, r * TH:r * TH + TH + 2]
            for p in range(4):
                ph, pw = p // 2, p % 2
                acc = jnp.dot(_patch(xbf, ph, pw, TH, W, Cin), wc_ref[p],
                              preferred_element_type=jnp.float32)
                s_tot = s_tot + jnp.sum(acc, axis=0, keepdims=True)
                ss_tot = ss_tot + jnp.sum(acc * acc, axis=0, keepdims=True)
        s_ref[b] = s_tot
        ss_ref[b] = ss_tot


def _apply_kernel(x_ref, wc_ref, s_ref, ss_ref, gam_ref, bet_ref, o_ref, scr,
                  *, TH, eps):
    NB = x_ref.shape[0]
    Hp = x_ref.shape[1]
    W = x_ref.shape[2] - 2
    Cin = x_ref.shape[3]
    Cout = wc_ref.shape[2]
    RT = (Hp - 2) // TH
    N = s_ref.shape[0]

    count = N * 4.0 * (Hp - 2) * W
    total = jnp.sum(s_ref[:, 0, :], axis=0, keepdims=True)
    total_sq = jnp.sum(ss_ref[:, 0, :], axis=0, keepdims=True)
    mean = total / count
    var = jnp.maximum(total_sq / count - mean * mean, 0.0)
    scale = gam_ref[...] * jax.lax.rsqrt(var + eps)
    shift = bet_ref[...] - mean * scale

    for b in range(NB):
        for r in range(RT):
            xbf = x_ref[b, r * TH:r * TH + TH + 2]
            for ph in range(2):
                zs = []
                for pw in range(2):
                    acc = jnp.dot(_patch(xbf, ph, pw, TH, W, Cin),
                                  wc_ref[ph * 2 + pw],
                                  preferred_element_type=jnp.float32)
                    zs.append(jnp.maximum(acc * scale + shift, 0.0)
                              .astype(jnp.bfloat16))
                zi = _pack_rows(zs[0], zs[1])
                zi3 = zi.reshape(TH, 2 * W, Cout)
                for a in range(TH):
                    scr[2 * a + ph] = zi3[a]
            for rr in range(2 * TH):
                o_ref[b, :, 2 * TH * r + rr, :] = (
                    jnp.transpose(scr[rr], (1, 0)).astype(jnp.float32))


def kernel(x, w, b, gamma, beta, eps=1e-5):
    del b
    N, Cin, H, W = x.shape
    Cout = w.shape[-1]
    TH = 16
    NB = 2
    NG = N // NB

    x_nhwc = jnp.transpose(x, (0, 2, 3, 1)).astype(jnp.bfloat16)
    x_p = jnp.pad(x_nhwc, ((0, 0), (1, 1), (1, 1), (0, 0)))
    wc = _parity_weights(w).astype(jnp.bfloat16)

    s, ss = pl.pallas_call(
        functools.partial(_stats_kernel, TH=TH),
        out_shape=(
            jax.ShapeDtypeStruct((N, 1, Cout), jnp.float32),
            jax.ShapeDtypeStruct((N, 1, Cout), jnp.float32),
        ),
        grid=(NG,),
        in_specs=[
            pl.BlockSpec((NB, H + 2, W + 2, Cin), lambda n: (n, 0, 0, 0)),
            pl.BlockSpec((4, 4 * Cin, Cout), lambda n: (0, 0, 0)),
        ],
        out_specs=(
            pl.BlockSpec((NB, 1, Cout), lambda n: (n, 0, 0)),
            pl.BlockSpec((NB, 1, Cout), lambda n: (n, 0, 0)),
        ),
        compiler_params=pltpu.CompilerParams(
            dimension_semantics=("parallel",),
            vmem_limit_bytes=64 * 1024 * 1024),
    )(x_p, wc)

    out = pl.pallas_call(
        functools.partial(_apply_kernel, TH=TH, eps=eps),
        out_shape=jax.ShapeDtypeStruct((N, Cout, 2 * H, 2 * W), jnp.float32),
        grid=(NG,),
        in_specs=[
            pl.BlockSpec((NB, H + 2, W + 2, Cin), lambda n: (n, 0, 0, 0)),
            pl.BlockSpec((4, 4 * Cin, Cout), lambda n: (0, 0, 0)),
            pl.BlockSpec((N, 1, Cout), lambda n: (0, 0, 0)),
            pl.BlockSpec((N, 1, Cout), lambda n: (0, 0, 0)),
            pl.BlockSpec((1, Cout), lambda n: (0, 0)),
            pl.BlockSpec((1, Cout), lambda n: (0, 0)),
        ],
        out_specs=pl.BlockSpec((NB, Cout, 2 * H, 2 * W),
                               lambda n: (n, 0, 0, 0)),
        scratch_shapes=[pltpu.VMEM((2 * TH, 2 * W, Cout), jnp.bfloat16)],
        compiler_params=pltpu.CompilerParams(
            dimension_semantics=("parallel",),
            vmem_limit_bytes=64 * 1024 * 1024),
    )(x_p, wc, s, ss, gamma.reshape(1, Cout), beta.reshape(1, Cout))
    return out
